# baseline (speedup 1.0000x reference)
import jax
import jax.numpy as jnp
from jax.experimental import pallas as pl


def _vq_body(z_ref, w_ref, ek_ref, ids_ref):
    zc = z_ref[0].reshape(z_ref.shape[1], -1)
    w = w_ref[...]
    K = w.shape[0]
    px = zc.shape[1]
    b2 = jax.lax.dot_general(w, zc, (((1,), (0,)), ((), ())),
                             preferred_element_type=jnp.float32)
    a = jnp.sum(zc * zc, axis=0)[None, :]
    c = jnp.sum(w * w, axis=1)[:, None]
    dist = (a - 2.0 * b2) + c
    iota = jax.lax.broadcasted_iota(jnp.int32, (K, px), 0)
    mval = jnp.min(dist, axis=0, keepdims=True)
    ids2 = jnp.min(jnp.where(dist == mval, iota, K), axis=0,
                   keepdims=True)
    onehot = (iota == ids2).astype(jnp.float32)
    ek = jax.lax.dot_general(w, onehot, (((0,), (0,)), ((), ())),
                             preferred_element_type=jnp.float32)
    ek_ref[0] = ek.reshape(ek_ref.shape[1:])
    ids_ref[0, 0] = ids2


def kernel(z_e, codebook):
    B, C, H, W = z_e.shape
    K = codebook.shape[0]
    ROWS = 8
    grid = (B, H // ROWS)
    ek, ids = pl.pallas_call(
        _vq_body,
        grid=grid,
        in_specs=[
            pl.BlockSpec((1, C, ROWS, W), lambda b, r: (b, 0, r, 0)),
            pl.BlockSpec((K, C), lambda b, r: (0, 0)),
        ],
        out_specs=[
            pl.BlockSpec((1, C, ROWS, W), lambda b, r: (b, 0, r, 0)),
            pl.BlockSpec((1, 1, 1, ROWS * W), lambda b, r: (b, r, 0, 0)),
        ],
        out_shape=[
            jax.ShapeDtypeStruct((B, C, H, W), jnp.float32),
            jax.ShapeDtypeStruct((B, H // ROWS, 1, ROWS * W), jnp.int32),
        ],
    )(z_e, codebook)
    return ek, ek, ids.reshape(B, H, W)


# speedup vs baseline: 1.9402x; 1.9402x over previous
import jax
import jax.numpy as jnp
from jax.experimental import pallas as pl
from jax.experimental.pallas import tpu as pltpu


def _vq_body(z_ref, w_ref, ek_ref, ids_ref, c_ref):
    w = w_ref[...]
    K = w.shape[0]

    @pl.when(pl.program_id(0) == 0)
    def _():
        c_ref[...] = jnp.sum(w * w, axis=1)[:, None]

    zc = z_ref[0]
    px = zc.shape[1]
    b2 = jax.lax.dot_general(w, zc, (((1,), (0,)), ((), ())),
                             preferred_element_type=jnp.float32)
    a = jnp.sum(zc * zc, axis=0)[None, :]
    dist = (a - 2.0 * b2) + c_ref[...]
    iota = jax.lax.broadcasted_iota(jnp.int32, (K, px), 0)
    mval = jnp.min(dist, axis=0, keepdims=True)
    ids2 = jnp.min(jnp.where(dist == mval, iota, K), axis=0,
                   keepdims=True)
    onehot = (iota == ids2).astype(jnp.float32)
    ek = jax.lax.dot_general(w, onehot, (((0,), (0,)), ((), ())),
                             preferred_element_type=jnp.float32)
    ek_ref[0] = ek
    ids_ref[0, 0] = ids2


def kernel(z_e, codebook):
    B, C, H, W = z_e.shape
    K = codebook.shape[0]
    P = H * W
    ek, ids = pl.pallas_call(
        _vq_body,
        grid=(B,),
        in_specs=[
            pl.BlockSpec((1, C, P), lambda b: (b, 0, 0)),
            pl.BlockSpec((K, C), lambda b: (0, 0)),
        ],
        out_specs=[
            pl.BlockSpec((1, C, P), lambda b: (b, 0, 0)),
            pl.BlockSpec((1, 1, 1, P), lambda b: (b, 0, 0, 0)),
        ],
        out_shape=[
            jax.ShapeDtypeStruct((B, C, P), jnp.float32),
            jax.ShapeDtypeStruct((B, 1, 1, P), jnp.int32),
        ],
        scratch_shapes=[pltpu.VMEM((K, 1), jnp.float32)],
    )(z_e.reshape(B, C, P), codebook)
    ek = ek.reshape(B, C, H, W)
    return ek, ek, ids.reshape(B, H, W)
